# baseline (speedup 1.0000x reference)
import functools

import jax
import jax.numpy as jnp
from jax import lax
from jax.experimental import pallas as pl
from jax.experimental.pallas import tpu as pltpu
from jax.experimental.pallas import tpu_sc as plsc

N_ENODES = 100000
N_ECLASSES = 50000
N_EDGES = 1600000
HIDDEN = 32

NC, NS = 2, 16
CHUNK = 250
ROWS_TOTAL = N_EDGES // CHUNK
ROWS_PER_SC = ROWS_TOTAL // NC
ROWS_PER_TILE = ROWS_PER_SC // NS
ROWS_PER_SLAB = 8
SLABS = ROWS_PER_TILE // ROWS_PER_SLAB
ACC_PAD = 50048
ACC_STRIPE = ACC_PAD // NS
CNT_PAD = ACC_PAD


PACK = 128 // HIDDEN
NP4 = N_ENODES // PACK


def _dense_body(emb_ref, w1b_ref, b1b_ref, gb_ref, beb_ref, w2b_ref, b2_ref,
                m_ref, mt_ref, h_ref, logit_ref):
    x = emb_ref[...]
    h = jnp.dot(x, w1b_ref[...], preferred_element_type=jnp.float32) + b1b_ref[...]
    m, mt = m_ref[...], mt_ref[...]
    mu4 = jnp.dot(h, m, preferred_element_type=jnp.float32) * (1.0 / HIDDEN)
    d = h - jnp.dot(mu4, mt, preferred_element_type=jnp.float32)
    var4 = jnp.dot(d * d, m, preferred_element_type=jnp.float32) * (1.0 / HIDDEN)
    var = jnp.dot(var4, mt, preferred_element_type=jnp.float32)
    h = gb_ref[...] * d * lax.rsqrt(var + 1e-5) + beb_ref[...]
    h = jnp.maximum(h, 0.0)
    h_ref[...] = h
    logit_ref[...] = (jnp.dot(h, w2b_ref[...], preferred_element_type=jnp.float32)
                      + b2_ref[...])


def _dense(emb4, W1, b1, gamma, beta, W2, b2):
    R = 5000
    grid = (NP4 // R,)
    eye4 = jnp.eye(PACK, dtype=jnp.float32)
    W1b = jnp.kron(eye4, W1)
    W2b = jnp.kron(eye4, W2)
    M = jnp.kron(eye4, jnp.ones((HIDDEN, 1), jnp.float32))
    b1b = jnp.tile(b1, PACK).reshape(1, -1)
    gb = jnp.tile(gamma, PACK).reshape(1, -1)
    beb = jnp.tile(beta, PACK).reshape(1, -1)
    h, logit = pl.pallas_call(
        _dense_body,
        grid=grid,
        in_specs=[
            pl.BlockSpec((R, 128), lambda i: (i, 0)),
            pl.BlockSpec((128, 128), lambda i: (0, 0)),
            pl.BlockSpec((1, 128), lambda i: (0, 0)),
            pl.BlockSpec((1, 128), lambda i: (0, 0)),
            pl.BlockSpec((1, 128), lambda i: (0, 0)),
            pl.BlockSpec((128, PACK), lambda i: (0, 0)),
            pl.BlockSpec((1, 1), lambda i: (0, 0)),
            pl.BlockSpec((128, PACK), lambda i: (0, 0)),
            pl.BlockSpec((PACK, 128), lambda i: (0, 0)),
        ],
        out_specs=[
            pl.BlockSpec((R, 128), lambda i: (i, 0)),
            pl.BlockSpec((R, PACK), lambda i: (i, 0)),
        ],
        out_shape=[
            jax.ShapeDtypeStruct((NP4, 128), jnp.float32),
            jax.ShapeDtypeStruct((NP4, PACK), jnp.float32),
        ],
    )(emb4, W1b, b1b, gb, beb, W2b, b2.reshape(1, 1), M, M.T)
    return h, logit


def _sc_body(h_ref, enode_ref, ecl_ref, z2d_ref, z1d_ref, ones_ref,
             acc_out, cnt_out,
             idx_slab, ecl_slab, rows0, rows1, ones_v, sem0, sem1,
             ssem0, ssem1, csem,
             acc_sp, cnt_sp):
    c = lax.axis_index("c")
    s = lax.axis_index("s")

    pltpu.sync_copy(ones_ref, ones_v)

    if True:
        pltpu.sync_copy(z2d_ref, acc_sp.at[pl.ds(s * ACC_STRIPE, ACC_STRIPE)])
        pltpu.sync_copy(z1d_ref, cnt_sp.at[pl.ds(s * ACC_STRIPE, ACC_STRIPE)])
        plsc.subcore_barrier()

        tile_row0 = c * ROWS_PER_SC + s * ROWS_PER_TILE

        bufs = (rows0, rows1)
        sems = (sem0, sem1)
        ssems = (ssem0, ssem1)

        def _slab(si, carry):
            row0 = tile_row0 + si * ROWS_PER_SLAB
            pltpu.sync_copy(enode_ref.at[pl.ds(row0, ROWS_PER_SLAB)], idx_slab)
            pltpu.sync_copy(ecl_ref.at[pl.ds(row0, ROWS_PER_SLAB)], ecl_slab)

            gd = [None, None]
            sd = [None, None]
            cds = []
            gd[0] = pltpu.async_copy(h_ref.at[idx_slab.at[0]], bufs[0], sems[0])
            for j in range(ROWS_PER_SLAB):
                p = j % 2
                gd[p].wait()
                sd[p] = pltpu.async_copy(bufs[p], acc_sp.at[ecl_slab.at[j]],
                                         ssems[p], add=True)
                cds.append(pltpu.async_copy(ones_v, cnt_sp.at[ecl_slab.at[j]],
                                            csem, add=True))
                if j + 1 < ROWS_PER_SLAB:
                    if sd[1 - p] is not None:
                        sd[1 - p].wait()
                    gd[1 - p] = pltpu.async_copy(
                        h_ref.at[idx_slab.at[j + 1]], bufs[1 - p], sems[1 - p])
            sd[0].wait()
            sd[1].wait()
            for d in cds:
                d.wait()
            return carry
        lax.fori_loop(0, SLABS, _slab, 0)

        plsc.subcore_barrier()

        r = s * ACC_STRIPE
        pltpu.sync_copy(acc_sp.at[pl.ds(r, ACC_STRIPE)],
                        acc_out.at[c, pl.ds(r, ACC_STRIPE)])
        pltpu.sync_copy(cnt_sp.at[pl.ds(r, ACC_STRIPE)],
                        cnt_out.at[c, pl.ds(r, ACC_STRIPE)])


def _segment_mean_partials(h, enode2d, ecl2d, z2d, z1d, ones2d):
    mesh = plsc.VectorSubcoreMesh(core_axis_name="c", subcore_axis_name="s",
                                  num_cores=NC, num_subcores=NS)
    fn = pl.kernel(
        _sc_body,
        out_type=[
            jax.ShapeDtypeStruct((NC, ACC_PAD, HIDDEN), jnp.float32),
            jax.ShapeDtypeStruct((NC, CNT_PAD), jnp.float32),
        ],
        mesh=mesh,
        scratch_types=[
            pltpu.VMEM((ROWS_PER_SLAB, CHUNK), jnp.int32),
            pltpu.VMEM((ROWS_PER_SLAB, CHUNK), jnp.int32),
            pltpu.VMEM((CHUNK, HIDDEN), jnp.float32),
            pltpu.VMEM((CHUNK, HIDDEN), jnp.float32),
            pltpu.VMEM((CHUNK,), jnp.float32),
            pltpu.SemaphoreType.DMA,
            pltpu.SemaphoreType.DMA,
            pltpu.SemaphoreType.DMA,
            pltpu.SemaphoreType.DMA,
            pltpu.SemaphoreType.DMA,
            pltpu.VMEM_SHARED((ACC_PAD, HIDDEN), jnp.float32),
            pltpu.VMEM_SHARED((CNT_PAD,), jnp.float32),
        ],
        compiler_params=pltpu.CompilerParams(use_tc_tiling_on_sc=False),
    )
    return fn(h, enode2d, ecl2d, z2d, z1d, ones2d)


NCTX4 = N_ECLASSES * HIDDEN // 128
NACC4 = ACC_PAD * HIDDEN // 128


def _combine_body(acc_ref, cnt_ref, mt_ref, ctx_ref):
    a = acc_ref[0] + acc_ref[1]
    n4 = cnt_ref[0] + cnt_ref[1]
    inv4 = 1.0 / jnp.maximum(n4, 1.0)
    inv = jnp.dot(inv4, mt_ref[...], preferred_element_type=jnp.float32)
    ctx_ref[...] = a * inv


def _combine(acc4, cnt4):
    R = NACC4 // 4
    grid = (4,)
    MT = jnp.kron(jnp.eye(PACK, dtype=jnp.float32),
                  jnp.ones((1, HIDDEN), jnp.float32))
    return pl.pallas_call(
        _combine_body,
        grid=grid,
        in_specs=[
            pl.BlockSpec((NC, R, 128), lambda i: (0, i, 0)),
            pl.BlockSpec((NC, R, PACK), lambda i: (0, i, 0)),
            pl.BlockSpec((PACK, 128), lambda i: (0, 0)),
        ],
        out_specs=pl.BlockSpec((R, 128), lambda i: (i, 0)),
        out_shape=jax.ShapeDtypeStruct((NCTX4, 128), jnp.float32),
    )(acc4, cnt4, MT)


def kernel(embedding, eclass_ids, enode_ids, W1, b1, gamma, beta, W2, b2):
    emb4 = embedding.reshape(NP4, 128)
    h4, logit4 = _dense(emb4, W1, b1, gamma, beta, W2, b2)
    h = h4.reshape(N_ENODES, HIDDEN)
    enode2d = enode_ids.reshape(ROWS_TOTAL, CHUNK)
    ecl2d = eclass_ids.reshape(ROWS_TOTAL, CHUNK)
    z2d = jnp.zeros((ACC_STRIPE, HIDDEN), jnp.float32)
    z1d = jnp.zeros((ACC_STRIPE,), jnp.float32)
    ones2d = jnp.ones((CHUNK,), jnp.float32)
    acc, cnt = _segment_mean_partials(h, enode2d, ecl2d, z2d, z1d, ones2d)
    ctx4 = _combine(acc.reshape(NC, NACC4, 128),
                    cnt.reshape(NC, NACC4, PACK))
    return logit4.reshape(1, N_ENODES), ctx4.reshape(1, N_ECLASSES, HIDDEN)


# speedup vs baseline: 1.0724x; 1.0724x over previous
import functools

import jax
import jax.numpy as jnp
from jax import lax
from jax.experimental import pallas as pl
from jax.experimental.pallas import tpu as pltpu
from jax.experimental.pallas import tpu_sc as plsc

N_ENODES = 100000
N_ECLASSES = 50000
N_EDGES = 1600000
HIDDEN = 32

NC, NS = 2, 16
CHUNK = 250
ROWS_TOTAL = N_EDGES // CHUNK
ROWS_PER_SC = ROWS_TOTAL // NC
ROWS_PER_TILE = ROWS_PER_SC // NS
ROWS_PER_SLAB = 8
SLABS = ROWS_PER_TILE // ROWS_PER_SLAB
ACC_PAD = 50048
ACC_STRIPE = ACC_PAD // NS
CNT_PAD = ACC_PAD


PACK = 128 // HIDDEN
NP4 = N_ENODES // PACK


def _dense_body(emb_ref, w1b_ref, b1b_ref, gb_ref, beb_ref, w2b_ref, b2_ref,
                m_ref, mt_ref, h_ref, logit_ref):
    x = emb_ref[...]
    h = jnp.dot(x, w1b_ref[...], preferred_element_type=jnp.float32) + b1b_ref[...]
    m, mt = m_ref[...], mt_ref[...]
    mu4 = jnp.dot(h, m, preferred_element_type=jnp.float32) * (1.0 / HIDDEN)
    d = h - jnp.dot(mu4, mt, preferred_element_type=jnp.float32)
    var4 = jnp.dot(d * d, m, preferred_element_type=jnp.float32) * (1.0 / HIDDEN)
    var = jnp.dot(var4, mt, preferred_element_type=jnp.float32)
    h = gb_ref[...] * d * lax.rsqrt(var + 1e-5) + beb_ref[...]
    h = jnp.maximum(h, 0.0)
    h_ref[...] = h
    logit_ref[...] = (jnp.dot(h, w2b_ref[...], preferred_element_type=jnp.float32)
                      + b2_ref[...])


def _dense(emb4, W1, b1, gamma, beta, W2, b2):
    R = 5000
    grid = (NP4 // R,)
    eye4 = jnp.eye(PACK, dtype=jnp.float32)
    W1b = jnp.kron(eye4, W1)
    W2b = jnp.kron(eye4, W2)
    M = jnp.kron(eye4, jnp.ones((HIDDEN, 1), jnp.float32))
    b1b = jnp.tile(b1, PACK).reshape(1, -1)
    gb = jnp.tile(gamma, PACK).reshape(1, -1)
    beb = jnp.tile(beta, PACK).reshape(1, -1)
    h, logit = pl.pallas_call(
        _dense_body,
        grid=grid,
        in_specs=[
            pl.BlockSpec((R, 128), lambda i: (i, 0)),
            pl.BlockSpec((128, 128), lambda i: (0, 0)),
            pl.BlockSpec((1, 128), lambda i: (0, 0)),
            pl.BlockSpec((1, 128), lambda i: (0, 0)),
            pl.BlockSpec((1, 128), lambda i: (0, 0)),
            pl.BlockSpec((128, PACK), lambda i: (0, 0)),
            pl.BlockSpec((1, 1), lambda i: (0, 0)),
            pl.BlockSpec((128, PACK), lambda i: (0, 0)),
            pl.BlockSpec((PACK, 128), lambda i: (0, 0)),
        ],
        out_specs=[
            pl.BlockSpec((R, 128), lambda i: (i, 0)),
            pl.BlockSpec((R, PACK), lambda i: (i, 0)),
        ],
        out_shape=[
            jax.ShapeDtypeStruct((NP4, 128), jnp.float32),
            jax.ShapeDtypeStruct((NP4, PACK), jnp.float32),
        ],
    )(emb4, W1b, b1b, gb, beb, W2b, b2.reshape(1, 1), M, M.T)
    return h, logit


def _sc_body(h_ref, enode_ref, ecl_ref, z2d_ref, z1d_ref, ones_ref,
             acc_out, cnt_out,
             idx_slab, ecl_slab, rows0, rows1, ones_v, sem0, sem1,
             ssem0, ssem1, csem,
             acc_sp, cnt_sp):
    c = lax.axis_index("c")
    s = lax.axis_index("s")

    pltpu.sync_copy(ones_ref, ones_v)

    if True:
        pltpu.sync_copy(z2d_ref, acc_sp.at[pl.ds(s * ACC_STRIPE, ACC_STRIPE)])
        pltpu.sync_copy(z1d_ref, cnt_sp.at[pl.ds(s * ACC_STRIPE, ACC_STRIPE)])
        plsc.subcore_barrier()

        tile_row0 = c * ROWS_PER_SC + s * ROWS_PER_TILE

        bufs = (rows0, rows1)
        sems = (sem0, sem1)
        ssems = (ssem0, ssem1)

        def _slab(si, carry):
            row0 = tile_row0 + si * ROWS_PER_SLAB
            pltpu.sync_copy(enode_ref.at[pl.ds(row0, ROWS_PER_SLAB)], idx_slab)
            pltpu.sync_copy(ecl_ref.at[pl.ds(row0, ROWS_PER_SLAB)], ecl_slab)

            gd = [None, None]
            cds = []
            gd[0] = pltpu.async_copy(h_ref.at[idx_slab.at[0]], bufs[0], sems[0])
            for j in range(ROWS_PER_SLAB):
                p = j % 2
                if j + 1 < ROWS_PER_SLAB:
                    gd[1 - p] = pltpu.async_copy(
                        h_ref.at[idx_slab.at[j + 1]], bufs[1 - p], sems[1 - p])
                gd[p].wait()
                pltpu.sync_copy(bufs[p], acc_sp.at[ecl_slab.at[j]], add=True)
                cds.append(pltpu.async_copy(ones_v, cnt_sp.at[ecl_slab.at[j]],
                                            csem, add=True))
            for d in cds:
                d.wait()
            return carry
        lax.fori_loop(0, SLABS, _slab, 0)

        plsc.subcore_barrier()

        r = s * ACC_STRIPE
        pltpu.sync_copy(acc_sp.at[pl.ds(r, ACC_STRIPE)],
                        acc_out.at[c, pl.ds(r, ACC_STRIPE)])
        pltpu.sync_copy(cnt_sp.at[pl.ds(r, ACC_STRIPE)],
                        cnt_out.at[c, pl.ds(r, ACC_STRIPE)])


def _segment_mean_partials(h, enode2d, ecl2d, z2d, z1d, ones2d):
    mesh = plsc.VectorSubcoreMesh(core_axis_name="c", subcore_axis_name="s",
                                  num_cores=NC, num_subcores=NS)
    fn = pl.kernel(
        _sc_body,
        out_type=[
            jax.ShapeDtypeStruct((NC, ACC_PAD, HIDDEN), jnp.float32),
            jax.ShapeDtypeStruct((NC, CNT_PAD), jnp.float32),
        ],
        mesh=mesh,
        scratch_types=[
            pltpu.VMEM((ROWS_PER_SLAB, CHUNK), jnp.int32),
            pltpu.VMEM((ROWS_PER_SLAB, CHUNK), jnp.int32),
            pltpu.VMEM((CHUNK, HIDDEN), jnp.float32),
            pltpu.VMEM((CHUNK, HIDDEN), jnp.float32),
            pltpu.VMEM((CHUNK,), jnp.float32),
            pltpu.SemaphoreType.DMA,
            pltpu.SemaphoreType.DMA,
            pltpu.SemaphoreType.DMA,
            pltpu.SemaphoreType.DMA,
            pltpu.SemaphoreType.DMA,
            pltpu.VMEM_SHARED((ACC_PAD, HIDDEN), jnp.float32),
            pltpu.VMEM_SHARED((CNT_PAD,), jnp.float32),
        ],
        compiler_params=pltpu.CompilerParams(use_tc_tiling_on_sc=False),
    )
    return fn(h, enode2d, ecl2d, z2d, z1d, ones2d)


NCTX4 = N_ECLASSES * HIDDEN // 128
NACC4 = ACC_PAD * HIDDEN // 128


def _combine_body(acc_ref, cnt_ref, mt_ref, ctx_ref):
    a = acc_ref[0] + acc_ref[1]
    n4 = cnt_ref[0] + cnt_ref[1]
    inv4 = 1.0 / jnp.maximum(n4, 1.0)
    inv = jnp.dot(inv4, mt_ref[...], preferred_element_type=jnp.float32)
    ctx_ref[...] = a * inv


def _combine(acc4, cnt4):
    R = NACC4 // 4
    grid = (4,)
    MT = jnp.kron(jnp.eye(PACK, dtype=jnp.float32),
                  jnp.ones((1, HIDDEN), jnp.float32))
    return pl.pallas_call(
        _combine_body,
        grid=grid,
        in_specs=[
            pl.BlockSpec((NC, R, 128), lambda i: (0, i, 0)),
            pl.BlockSpec((NC, R, PACK), lambda i: (0, i, 0)),
            pl.BlockSpec((PACK, 128), lambda i: (0, 0)),
        ],
        out_specs=pl.BlockSpec((R, 128), lambda i: (i, 0)),
        out_shape=jax.ShapeDtypeStruct((NCTX4, 128), jnp.float32),
    )(acc4, cnt4, MT)


def kernel(embedding, eclass_ids, enode_ids, W1, b1, gamma, beta, W2, b2):
    emb4 = embedding.reshape(NP4, 128)
    h4, logit4 = _dense(emb4, W1, b1, gamma, beta, W2, b2)
    h = h4.reshape(N_ENODES, HIDDEN)
    enode2d = enode_ids.reshape(ROWS_TOTAL, CHUNK)
    ecl2d = eclass_ids.reshape(ROWS_TOTAL, CHUNK)
    z2d = jnp.zeros((ACC_STRIPE, HIDDEN), jnp.float32)
    z1d = jnp.zeros((ACC_STRIPE,), jnp.float32)
    ones2d = jnp.ones((CHUNK,), jnp.float32)
    acc, cnt = _segment_mean_partials(h, enode2d, ecl2d, z2d, z1d, ones2d)
    ctx4 = _combine(acc.reshape(NC, NACC4, 128),
                    cnt.reshape(NC, NACC4, PACK))
    return logit4.reshape(1, N_ENODES), ctx4.reshape(1, N_ECLASSES, HIDDEN)


# speedup vs baseline: 1.1531x; 1.0752x over previous
import functools

import jax
import jax.numpy as jnp
from jax import lax
from jax.experimental import pallas as pl
from jax.experimental.pallas import tpu as pltpu
from jax.experimental.pallas import tpu_sc as plsc

N_ENODES = 100000
N_ECLASSES = 50000
N_EDGES = 1600000
HIDDEN = 32

NC, NS = 2, 16
CHUNK = 200
EDGES_PER_TILE = N_EDGES // (NC * NS)
ROWS_PER_SLAB = 10
SLAB_EDGES = ROWS_PER_SLAB * CHUNK
SLABS = EDGES_PER_TILE // SLAB_EDGES
ACC_PAD = 50048
ACC_STRIPE = ACC_PAD // NS
CNT_PAD = ACC_PAD


PACK = 128 // HIDDEN
NP4 = N_ENODES // PACK


def _dense_body(emb_ref, w1b_ref, b1b_ref, gb_ref, beb_ref, w2b_ref, b2_ref,
                m_ref, mt_ref, h_ref, logit_ref):
    x = emb_ref[...]
    h = jnp.dot(x, w1b_ref[...], preferred_element_type=jnp.float32) + b1b_ref[...]
    m, mt = m_ref[...], mt_ref[...]
    mu4 = jnp.dot(h, m, preferred_element_type=jnp.float32) * (1.0 / HIDDEN)
    d = h - jnp.dot(mu4, mt, preferred_element_type=jnp.float32)
    var4 = jnp.dot(d * d, m, preferred_element_type=jnp.float32) * (1.0 / HIDDEN)
    var = jnp.dot(var4, mt, preferred_element_type=jnp.float32)
    h = gb_ref[...] * d * lax.rsqrt(var + 1e-5) + beb_ref[...]
    h = jnp.maximum(h, 0.0)
    h_ref[...] = h
    logit_ref[...] = (jnp.dot(h, w2b_ref[...], preferred_element_type=jnp.float32)
                      + b2_ref[...])


def _dense(emb4, W1, b1, gamma, beta, W2, b2):
    R = 5000
    grid = (NP4 // R,)
    eye4 = jnp.eye(PACK, dtype=jnp.float32)
    W1b = jnp.kron(eye4, W1)
    W2b = jnp.kron(eye4, W2)
    M = jnp.kron(eye4, jnp.ones((HIDDEN, 1), jnp.float32))
    b1b = jnp.tile(b1, PACK).reshape(1, -1)
    gb = jnp.tile(gamma, PACK).reshape(1, -1)
    beb = jnp.tile(beta, PACK).reshape(1, -1)
    h, logit = pl.pallas_call(
        _dense_body,
        grid=grid,
        in_specs=[
            pl.BlockSpec((R, 128), lambda i: (i, 0)),
            pl.BlockSpec((128, 128), lambda i: (0, 0)),
            pl.BlockSpec((1, 128), lambda i: (0, 0)),
            pl.BlockSpec((1, 128), lambda i: (0, 0)),
            pl.BlockSpec((1, 128), lambda i: (0, 0)),
            pl.BlockSpec((128, PACK), lambda i: (0, 0)),
            pl.BlockSpec((1, 1), lambda i: (0, 0)),
            pl.BlockSpec((128, PACK), lambda i: (0, 0)),
            pl.BlockSpec((PACK, 128), lambda i: (0, 0)),
        ],
        out_specs=[
            pl.BlockSpec((R, 128), lambda i: (i, 0)),
            pl.BlockSpec((R, PACK), lambda i: (i, 0)),
        ],
        out_shape=[
            jax.ShapeDtypeStruct((NP4, 128), jnp.float32),
            jax.ShapeDtypeStruct((NP4, PACK), jnp.float32),
        ],
    )(emb4, W1b, b1b, gb, beb, W2b, b2.reshape(1, 1), M, M.T)
    return h, logit


def _sc_body(h_ref, enode_ref, ecl_ref, z2d_ref, z1d_ref, ones_ref,
             acc_out, cnt_out,
             idx_slab, ecl_slab, rows0, rows1, ones_v, sem0, sem1,
             ssem0, ssem1, csem,
             acc_sp, cnt_sp):
    c = lax.axis_index("c")
    s = lax.axis_index("s")

    pltpu.sync_copy(ones_ref, ones_v)

    if True:
        pltpu.sync_copy(z2d_ref, acc_sp.at[pl.ds(s * ACC_STRIPE, ACC_STRIPE)])
        pltpu.sync_copy(z1d_ref, cnt_sp.at[pl.ds(s * ACC_STRIPE, ACC_STRIPE)])
        plsc.subcore_barrier()

        tile_e0 = (c * NS + s) * EDGES_PER_TILE

        bufs = (rows0, rows1)
        sems = (sem0, sem1)
        ssems = (ssem0, ssem1)

        def _slab(si, carry):
            e0 = tile_e0 + si * SLAB_EDGES
            pltpu.sync_copy(enode_ref.at[pl.ds(e0, ROWS_PER_SLAB * CHUNK)], idx_slab)
            pltpu.sync_copy(ecl_ref.at[pl.ds(e0, ROWS_PER_SLAB * CHUNK)], ecl_slab)

            gd = [None, None]
            cds = []
            gd[0] = pltpu.async_copy(
                h_ref.at[idx_slab.at[pl.ds(0, CHUNK)]], bufs[0], sems[0])
            for j in range(ROWS_PER_SLAB):
                p = j % 2
                if j + 1 < ROWS_PER_SLAB:
                    gd[1 - p] = pltpu.async_copy(
                        h_ref.at[idx_slab.at[pl.ds((j + 1) * CHUNK, CHUNK)]],
                        bufs[1 - p], sems[1 - p])
                gd[p].wait()
                pltpu.sync_copy(bufs[p],
                                acc_sp.at[ecl_slab.at[pl.ds(j * CHUNK, CHUNK)]],
                                add=True)
                cds.append(pltpu.async_copy(
                    ones_v, cnt_sp.at[ecl_slab.at[pl.ds(j * CHUNK, CHUNK)]],
                    csem, add=True))
            for d in cds:
                d.wait()
            return carry
        lax.fori_loop(0, SLABS, _slab, 0)

        plsc.subcore_barrier()

        r = s * ACC_STRIPE
        pltpu.sync_copy(acc_sp.at[pl.ds(r, ACC_STRIPE)],
                        acc_out.at[c, pl.ds(r, ACC_STRIPE)])
        pltpu.sync_copy(cnt_sp.at[pl.ds(r, ACC_STRIPE)],
                        cnt_out.at[c, pl.ds(r, ACC_STRIPE)])


def _segment_mean_partials(h, enode2d, ecl2d, z2d, z1d, ones2d):
    mesh = plsc.VectorSubcoreMesh(core_axis_name="c", subcore_axis_name="s",
                                  num_cores=NC, num_subcores=NS)
    fn = pl.kernel(
        _sc_body,
        out_type=[
            jax.ShapeDtypeStruct((NC, ACC_PAD, HIDDEN), jnp.float32),
            jax.ShapeDtypeStruct((NC, CNT_PAD), jnp.float32),
        ],
        mesh=mesh,
        scratch_types=[
            pltpu.VMEM((SLAB_EDGES,), jnp.int32),
            pltpu.VMEM((SLAB_EDGES,), jnp.int32),
            pltpu.VMEM((CHUNK, HIDDEN), jnp.float32),
            pltpu.VMEM((CHUNK, HIDDEN), jnp.float32),
            pltpu.VMEM((CHUNK,), jnp.float32),
            pltpu.SemaphoreType.DMA,
            pltpu.SemaphoreType.DMA,
            pltpu.SemaphoreType.DMA,
            pltpu.SemaphoreType.DMA,
            pltpu.SemaphoreType.DMA,
            pltpu.VMEM_SHARED((ACC_PAD, HIDDEN), jnp.float32),
            pltpu.VMEM_SHARED((CNT_PAD,), jnp.float32),
        ],
        compiler_params=pltpu.CompilerParams(use_tc_tiling_on_sc=False),
    )
    return fn(h, enode2d, ecl2d, z2d, z1d, ones2d)


NCTX4 = N_ECLASSES * HIDDEN // 128
NACC4 = ACC_PAD * HIDDEN // 128


def _combine_body(acc_ref, cnt_ref, mt_ref, ctx_ref):
    a = acc_ref[0] + acc_ref[1]
    n4 = cnt_ref[0] + cnt_ref[1]
    inv4 = 1.0 / jnp.maximum(n4, 1.0)
    inv = jnp.dot(inv4, mt_ref[...], preferred_element_type=jnp.float32)
    ctx_ref[...] = a * inv


def _combine(acc4, cnt4):
    R = NACC4 // 4
    grid = (4,)
    MT = jnp.kron(jnp.eye(PACK, dtype=jnp.float32),
                  jnp.ones((1, HIDDEN), jnp.float32))
    return pl.pallas_call(
        _combine_body,
        grid=grid,
        in_specs=[
            pl.BlockSpec((NC, R, 128), lambda i: (0, i, 0)),
            pl.BlockSpec((NC, R, PACK), lambda i: (0, i, 0)),
            pl.BlockSpec((PACK, 128), lambda i: (0, 0)),
        ],
        out_specs=pl.BlockSpec((R, 128), lambda i: (i, 0)),
        out_shape=jax.ShapeDtypeStruct((NCTX4, 128), jnp.float32),
    )(acc4, cnt4, MT)


def kernel(embedding, eclass_ids, enode_ids, W1, b1, gamma, beta, W2, b2):
    emb4 = embedding.reshape(NP4, 128)
    h4, logit4 = _dense(emb4, W1, b1, gamma, beta, W2, b2)
    h = h4.reshape(N_ENODES, HIDDEN)
    z2d = jnp.zeros((ACC_STRIPE, HIDDEN), jnp.float32)
    z1d = jnp.zeros((ACC_STRIPE,), jnp.float32)
    ones2d = jnp.ones((CHUNK,), jnp.float32)
    acc, cnt = _segment_mean_partials(h, enode_ids, eclass_ids, z2d, z1d, ones2d)
    ctx4 = _combine(acc.reshape(NC, NACC4, 128),
                    cnt.reshape(NC, NACC4, PACK))
    return logit4.reshape(1, N_ENODES), ctx4.reshape(1, N_ECLASSES, HIDDEN)


# speedup vs baseline: 1.2499x; 1.0840x over previous
import functools

import jax
import jax.numpy as jnp
from jax import lax
from jax.experimental import pallas as pl
from jax.experimental.pallas import tpu as pltpu
from jax.experimental.pallas import tpu_sc as plsc

N_ENODES = 100000
N_ECLASSES = 50000
N_EDGES = 1600000
HIDDEN = 32

NC, NS = 2, 16
CHUNK = 200
EDGES_PER_TILE = N_EDGES // (NC * NS)
CHUNKS_PER_SLAB = 5
SLAB_EDGES = CHUNKS_PER_SLAB * CHUNK
SLABS = EDGES_PER_TILE // SLAB_EDGES
ACC_PAD = 50048
ACC_STRIPE = ACC_PAD // NS
CNT_PAD = ACC_PAD


PACK = 128 // HIDDEN
NP4 = N_ENODES // PACK


def _dense_body(emb_ref, w1b_ref, b1b_ref, gb_ref, beb_ref, w2b_ref, b2_ref,
                m_ref, mt_ref, h_ref, logit_ref):
    x = emb_ref[...]
    h = jnp.dot(x, w1b_ref[...], preferred_element_type=jnp.float32) + b1b_ref[...]
    m, mt = m_ref[...], mt_ref[...]
    mu4 = jnp.dot(h, m, preferred_element_type=jnp.float32) * (1.0 / HIDDEN)
    d = h - jnp.dot(mu4, mt, preferred_element_type=jnp.float32)
    var4 = jnp.dot(d * d, m, preferred_element_type=jnp.float32) * (1.0 / HIDDEN)
    var = jnp.dot(var4, mt, preferred_element_type=jnp.float32)
    h = gb_ref[...] * d * lax.rsqrt(var + 1e-5) + beb_ref[...]
    h = jnp.maximum(h, 0.0)
    h_ref[...] = h
    logit_ref[...] = (jnp.dot(h, w2b_ref[...], preferred_element_type=jnp.float32)
                      + b2_ref[...])


def _dense(emb4, W1, b1, gamma, beta, W2, b2):
    R = 5000
    grid = (NP4 // R,)
    eye4 = jnp.eye(PACK, dtype=jnp.float32)
    W1b = jnp.kron(eye4, W1)
    W2b = jnp.kron(eye4, W2)
    M = jnp.kron(eye4, jnp.ones((HIDDEN, 1), jnp.float32))
    b1b = jnp.tile(b1, PACK).reshape(1, -1)
    gb = jnp.tile(gamma, PACK).reshape(1, -1)
    beb = jnp.tile(beta, PACK).reshape(1, -1)
    h, logit = pl.pallas_call(
        _dense_body,
        grid=grid,
        in_specs=[
            pl.BlockSpec((R, 128), lambda i: (i, 0)),
            pl.BlockSpec((128, 128), lambda i: (0, 0)),
            pl.BlockSpec((1, 128), lambda i: (0, 0)),
            pl.BlockSpec((1, 128), lambda i: (0, 0)),
            pl.BlockSpec((1, 128), lambda i: (0, 0)),
            pl.BlockSpec((128, PACK), lambda i: (0, 0)),
            pl.BlockSpec((1, 1), lambda i: (0, 0)),
            pl.BlockSpec((128, PACK), lambda i: (0, 0)),
            pl.BlockSpec((PACK, 128), lambda i: (0, 0)),
        ],
        out_specs=[
            pl.BlockSpec((R, 128), lambda i: (i, 0)),
            pl.BlockSpec((R, PACK), lambda i: (i, 0)),
        ],
        out_shape=[
            jax.ShapeDtypeStruct((NP4, 128), jnp.float32),
            jax.ShapeDtypeStruct((NP4, PACK), jnp.float32),
        ],
    )(emb4, W1b, b1b, gb, beb, W2b, b2.reshape(1, 1), M, M.T)
    return h, logit


def _sc_body(h_ref, enode_ref, ecl_ref, z2d_ref, z1d_ref, ones_ref,
             acc_out, cnt_out,
             idxA, idxB, eclA, eclB, rows0, rows1, ones_v,
             sem0, sem1, lsemA, lsemB, csem,
             acc_sp, cnt_sp):
    c = lax.axis_index("c")
    s = lax.axis_index("s")

    pltpu.sync_copy(ones_ref, ones_v)

    if True:
        pltpu.sync_copy(z2d_ref, acc_sp.at[pl.ds(s * ACC_STRIPE, ACC_STRIPE)])
        pltpu.sync_copy(z1d_ref, cnt_sp.at[pl.ds(s * ACC_STRIPE, ACC_STRIPE)])
        plsc.subcore_barrier()

        tile_e0 = (c * NS + s) * EDGES_PER_TILE
        e_last = N_EDGES - SLAB_EDGES

        bufs = (rows0, rows1)
        sems = (sem0, sem1)

        pltpu.sync_copy(enode_ref.at[pl.ds(tile_e0, SLAB_EDGES)], idxA)
        pltpu.sync_copy(ecl_ref.at[pl.ds(tile_e0, SLAB_EDGES)], eclA)
        pltpu.async_copy(h_ref.at[idxA.at[pl.ds(0, CHUNK)]], bufs[0], sems[0])

        def _body(b, carry):
            base = tile_e0 + (2 * b) * SLAB_EDGES
            cds = []

            def half(idx_cur, ecl_cur, idx_nxt, ecl_nxt, lsem_nxt, nxt_e0,
                     chunk0):
                eN = jnp.minimum(nxt_e0, e_last)
                ldi = pltpu.async_copy(enode_ref.at[pl.ds(eN, SLAB_EDGES)],
                                       idx_nxt, lsem_nxt)
                lde = pltpu.async_copy(ecl_ref.at[pl.ds(eN, SLAB_EDGES)],
                                       ecl_nxt, lsem_nxt)
                for jj in range(CHUNKS_PER_SLAB):
                    gp = (chunk0 + jj) % 2
                    if jj + 1 < CHUNKS_PER_SLAB:
                        pltpu.async_copy(
                            h_ref.at[idx_cur.at[pl.ds((jj + 1) * CHUNK, CHUNK)]],
                            bufs[1 - gp], sems[1 - gp])
                    else:
                        ldi.wait()
                        lde.wait()
                        pltpu.async_copy(
                            h_ref.at[idx_nxt.at[pl.ds(0, CHUNK)]],
                            bufs[1 - gp], sems[1 - gp])
                    pltpu.make_async_copy(
                        h_ref.at[idx_cur.at[pl.ds(jj * CHUNK, CHUNK)]],
                        bufs[gp], sems[gp]).wait()
                    cds.append(pltpu.async_copy(
                        ones_v,
                        cnt_sp.at[ecl_cur.at[pl.ds(jj * CHUNK, CHUNK)]],
                        csem, add=True))
                    pltpu.sync_copy(
                        bufs[gp],
                        acc_sp.at[ecl_cur.at[pl.ds(jj * CHUNK, CHUNK)]],
                        add=True)

            half(idxA, eclA, idxB, eclB, lsemB, base + SLAB_EDGES, 0)
            half(idxB, eclB, idxA, eclA, lsemA, base + 2 * SLAB_EDGES,
                 CHUNKS_PER_SLAB)
            for d in cds:
                d.wait()
            return carry
        lax.fori_loop(0, SLABS // 2, _body, 0)

        pltpu.make_async_copy(
            h_ref.at[idxA.at[pl.ds(0, CHUNK)]], bufs[0], sems[0]).wait()

        plsc.subcore_barrier()

        r = s * ACC_STRIPE
        pltpu.sync_copy(acc_sp.at[pl.ds(r, ACC_STRIPE)],
                        acc_out.at[c, pl.ds(r, ACC_STRIPE)])
        pltpu.sync_copy(cnt_sp.at[pl.ds(r, ACC_STRIPE)],
                        cnt_out.at[c, pl.ds(r, ACC_STRIPE)])


def _segment_mean_partials(h, enode2d, ecl2d, z2d, z1d, ones2d):
    mesh = plsc.VectorSubcoreMesh(core_axis_name="c", subcore_axis_name="s",
                                  num_cores=NC, num_subcores=NS)
    fn = pl.kernel(
        _sc_body,
        out_type=[
            jax.ShapeDtypeStruct((NC, ACC_PAD, HIDDEN), jnp.float32),
            jax.ShapeDtypeStruct((NC, CNT_PAD), jnp.float32),
        ],
        mesh=mesh,
        scratch_types=[
            pltpu.VMEM((SLAB_EDGES,), jnp.int32),
            pltpu.VMEM((SLAB_EDGES,), jnp.int32),
            pltpu.VMEM((SLAB_EDGES,), jnp.int32),
            pltpu.VMEM((SLAB_EDGES,), jnp.int32),
            pltpu.VMEM((CHUNK, HIDDEN), jnp.float32),
            pltpu.VMEM((CHUNK, HIDDEN), jnp.float32),
            pltpu.VMEM((CHUNK,), jnp.float32),
            pltpu.SemaphoreType.DMA,
            pltpu.SemaphoreType.DMA,
            pltpu.SemaphoreType.DMA,
            pltpu.SemaphoreType.DMA,
            pltpu.SemaphoreType.DMA,
            pltpu.VMEM_SHARED((ACC_PAD, HIDDEN), jnp.float32),
            pltpu.VMEM_SHARED((CNT_PAD,), jnp.float32),
        ],
        compiler_params=pltpu.CompilerParams(use_tc_tiling_on_sc=False),
    )
    return fn(h, enode2d, ecl2d, z2d, z1d, ones2d)


NCTX4 = N_ECLASSES * HIDDEN // 128
NACC4 = ACC_PAD * HIDDEN // 128


def _combine_body(acc_ref, cnt_ref, mt_ref, ctx_ref):
    a = acc_ref[0] + acc_ref[1]
    n4 = cnt_ref[0] + cnt_ref[1]
    inv4 = 1.0 / jnp.maximum(n4, 1.0)
    inv = jnp.dot(inv4, mt_ref[...], preferred_element_type=jnp.float32)
    ctx_ref[...] = a * inv


def _combine(acc4, cnt4):
    R = NACC4 // 4
    grid = (4,)
    MT = jnp.kron(jnp.eye(PACK, dtype=jnp.float32),
                  jnp.ones((1, HIDDEN), jnp.float32))
    return pl.pallas_call(
        _combine_body,
        grid=grid,
        in_specs=[
            pl.BlockSpec((NC, R, 128), lambda i: (0, i, 0)),
            pl.BlockSpec((NC, R, PACK), lambda i: (0, i, 0)),
            pl.BlockSpec((PACK, 128), lambda i: (0, 0)),
        ],
        out_specs=pl.BlockSpec((R, 128), lambda i: (i, 0)),
        out_shape=jax.ShapeDtypeStruct((NCTX4, 128), jnp.float32),
    )(acc4, cnt4, MT)


def kernel(embedding, eclass_ids, enode_ids, W1, b1, gamma, beta, W2, b2):
    emb4 = embedding.reshape(NP4, 128)
    h4, logit4 = _dense(emb4, W1, b1, gamma, beta, W2, b2)
    h = h4.reshape(N_ENODES, HIDDEN)
    z2d = jnp.zeros((ACC_STRIPE, HIDDEN), jnp.float32)
    z1d = jnp.zeros((ACC_STRIPE,), jnp.float32)
    ones2d = jnp.ones((CHUNK,), jnp.float32)
    acc, cnt = _segment_mean_partials(h, enode_ids, eclass_ids, z2d, z1d, ones2d)
    ctx4 = _combine(acc.reshape(NC, NACC4, 128),
                    cnt.reshape(NC, NACC4, PACK))
    return logit4.reshape(1, N_ENODES), ctx4.reshape(1, N_ECLASSES, HIDDEN)
